# baseline (speedup 1.0000x reference)
import functools
import math

import jax
import jax.numpy as jnp
from jax import lax
from jax.experimental import pallas as pl
from jax.experimental.pallas import tpu as pltpu
from jax.experimental.pallas import tpu_sc as plsc

_MARGIN = 0.5
_SCALE = 64.0
_COS_M = math.cos(_MARGIN)
_SIN_M = math.sin(_MARGIN)
_OVERFLOW_THRESH = -math.cos(_MARGIN)
_C1 = _SCALE * math.log2(math.e)
_NEG_HUGE = -1e30

_ROWS_SC = 384
_CB = 2048


def _margined(t):
    tm = t * _COS_M - jnp.sqrt(jnp.maximum(1.0 - t * t, 0.0)) * _SIN_M
    return jnp.where(t < _OVERFLOW_THRESH, t, tm)


def _sc_gather_body(nclass, chunk, flat_ref, y_ref, t_ref, y_v, idx_v, t_v, sem):
    wid = lax.axis_index("s") * 2 + lax.axis_index("c")
    base = wid * chunk
    pltpu.sync_copy(y_ref.at[pl.ds(base, chunk)], y_v)
    for c in range(chunk // 16):
        row = base + c * 16 + lax.iota(jnp.int32, 16)
        idx_v[pl.ds(c * 16, 16)] = row * nclass + y_v[pl.ds(c * 16, 16)]
    pltpu.async_copy(flat_ref.at[idx_v], t_v, sem).wait()
    pltpu.sync_copy(t_v, t_ref.at[pl.ds(base, chunk)])


def _sc_gather(y_hat_flat, y, b, n):
    chunk = b // 32
    mesh = plsc.VectorSubcoreMesh(core_axis_name="c", subcore_axis_name="s")
    kfn = functools.partial(
        pl.kernel,
        mesh=mesh,
        out_type=jax.ShapeDtypeStruct((b,), jnp.float32),
        scratch_types=[
            pltpu.VMEM((chunk,), jnp.int32),
            pltpu.VMEM((chunk,), jnp.int32),
            pltpu.VMEM((chunk,), jnp.float32),
            pltpu.SemaphoreType.DMA,
        ],
    )(functools.partial(_sc_gather_body, n, chunk))
    return kfn(y_hat_flat, y)


def _sc_sumexp_body(row0, rows_per_w, n, flat_ref, out_ref, x0_v, x1_v, o_v,
                    sem0, sem1):
    wid = lax.axis_index("s") * 2 + lax.axis_index("c")
    half = n // 2
    r0 = row0 + wid * rows_per_w
    bufs = (x0_v, x1_v)
    sems = (sem0, sem1)
    nchunks = rows_per_w * 2
    handles = {}

    def _start(c):
        row = r0 + (c // 2)
        off = row * n + (c % 2) * half
        handles[c] = pltpu.async_copy(flat_ref.at[pl.ds(off, half)],
                                      bufs[c % 2], sems[c % 2])

    _start(0)
    for c in range(nchunks):
        if c + 1 < nchunks:
            _start(c + 1)
        buf = bufs[c % 2]
        handles.pop(c).wait()

        def _inner(i, acc, buf=buf):
            base = i * 80
            for u in range(5):
                x = buf[pl.ds(base + u * 16, 16)]
                acc = acc + jnp.exp(x * _SCALE - _SCALE)
            return acc

        init = jnp.zeros((16,), jnp.float32) if c % 2 == 0 else o_v[...]
        acc = lax.fori_loop(0, half // 80, _inner, init)
        o_v[...] = acc
        if c % 2 == 1:
            pltpu.sync_copy(o_v, out_ref.at[(c - 1) // 2 + wid * rows_per_w])


def _sc_sumexp(y_hat_flat, row0, rows_sc, n):
    rows_per_w = rows_sc // 32
    half = n // 2
    mesh = plsc.VectorSubcoreMesh(core_axis_name="c", subcore_axis_name="s")
    kfn = functools.partial(
        pl.kernel,
        mesh=mesh,
        out_type=jax.ShapeDtypeStruct((rows_sc, 16), jnp.float32),
        scratch_types=[
            pltpu.VMEM((half,), jnp.float32),
            pltpu.VMEM((half,), jnp.float32),
            pltpu.VMEM((16,), jnp.float32),
            pltpu.SemaphoreType.DMA,
            pltpu.SemaphoreType.DMA,
        ],
    )(functools.partial(_sc_sumexp_body, row0, rows_per_w, n))
    return kfn(y_hat_flat)


def _tc_body(x_ref, out_ref, acc_ref, *, ncb, nclass, cb):
    j = pl.program_id(0)

    @pl.when(j == 0)
    def _init():
        acc_ref[...] = jnp.zeros_like(acc_ref)

    x = x_ref[...]
    col = j * cb + lax.broadcasted_iota(jnp.int32, x.shape, 1)
    z = jnp.where(col < nclass, x * _C1 - _C1, _NEG_HUGE)
    acc_ref[...] = acc_ref[...] + jnp.sum(jnp.exp2(z), axis=1, keepdims=True)

    @pl.when(j == ncb - 1)
    def _fin():
        out_ref[...] = acc_ref[...]


def _tc_sumexp(y_hat, rows_tc, cb):
    b, n = y_hat.shape
    ncb = pl.cdiv(n, cb)
    return pl.pallas_call(
        functools.partial(_tc_body, ncb=ncb, nclass=n, cb=cb),
        grid=(ncb,),
        in_specs=[pl.BlockSpec((rows_tc, cb), lambda j: (0, j))],
        out_specs=pl.BlockSpec((rows_tc, 1), lambda j: (0, 0)),
        out_shape=jax.ShapeDtypeStruct((rows_tc, 1), jnp.float32),
        scratch_shapes=[pltpu.VMEM((rows_tc, 1), jnp.float32)],
    )(y_hat)


def _ep_body(stc_ref, ssc_ref, t_ref, out_ref, *, batch):
    s_sc = jnp.sum(ssc_ref[...], axis=1, keepdims=True)
    s = jnp.concatenate([stc_ref[...], s_sc], axis=0)
    t = t_ref[...]
    tm = _margined(t)
    e_t = jnp.exp2(t * _C1 - _C1)
    e_tm = jnp.exp2(tm * _C1 - _C1)
    s_mod = s - e_t + e_tm
    loss_rows = jnp.log(s_mod) + (_SCALE - _SCALE * tm)
    out_ref[...] = jnp.sum(loss_rows, axis=(0, 1), keepdims=True) / batch


def _epilogue(s_tc, s_sc_l, t, b):
    out = pl.pallas_call(
        functools.partial(_ep_body, batch=b),
        out_shape=jax.ShapeDtypeStruct((1, 1), jnp.float32),
    )(s_tc, s_sc_l, t.reshape(b, 1))
    return out[0, 0]


@jax.jit
def kernel(y_hat, y):
    b, n = y_hat.shape
    flat = y_hat.reshape(b * n)
    rows_tc = b - _ROWS_SC
    t = _sc_gather(flat, y, b, n)
    s_sc_l = _sc_sumexp(flat, rows_tc, _ROWS_SC, n)
    s_tc = _tc_sumexp(y_hat, rows_tc, _CB)
    return _epilogue(s_tc, s_sc_l, t, b)


# speedup vs baseline: 1.9456x; 1.9456x over previous
import functools
import math

import jax
import jax.numpy as jnp
from jax import lax
from jax.experimental import pallas as pl
from jax.experimental.pallas import tpu as pltpu
from jax.experimental.pallas import tpu_sc as plsc

_MARGIN = 0.5
_SCALE = 64.0
_COS_M = math.cos(_MARGIN)
_SIN_M = math.sin(_MARGIN)
_OVERFLOW_THRESH = -math.cos(_MARGIN)
_C1 = _SCALE * math.log2(math.e)
_NEG_HUGE = -1e30

_ROWS_SC = 512
_CB = 2048
_PW = 1408
_NP = 71
_NCOV = _PW * _NP


def _margined(t):
    tm = t * _COS_M - jnp.sqrt(jnp.maximum(1.0 - t * t, 0.0)) * _SIN_M
    return jnp.where(t < _OVERFLOW_THRESH, t, tm)


def _sc_sumexp_body(row0, n, x_ref, y_ref, s_out, t_out,
                    x0_v, x1_v, y_v, o2_v, t2_v, sem0, sem1):
    wid = lax.axis_index("s") * 2 + lax.axis_index("c")
    r0 = row0 + wid * 16
    bufs = (x0_v, x1_v)
    sems = (sem0, sem1)

    pltpu.sync_copy(y_ref.at[pl.ds(r0, 16)], y_v)
    y_all = y_v[...]
    lane = lax.iota(jnp.int32, 16)

    for g in range(2):
        rg = pl.multiple_of(r0 + g * 8, 8)
        def _copy(p, buf, sem, rg=rg):
            c0 = pl.multiple_of(p * _PW, 128)
            return pltpu.make_async_copy(
                x_ref.at[pl.ds(rg, 8), pl.ds(c0, _PW)], buf, sem)

        o2_v[...] = jnp.zeros((8, 16), jnp.float32)
        t2_v[...] = jnp.zeros((8, 16), jnp.float32)
        _copy(0, x0_v, sem0).start()

        def _compute(buf, p, g=g):
            for r in range(8):
                acc = o2_v[r]

                def _inner(i, acc, buf=buf, r=r):
                    base = i * 64
                    for u in range(4):
                        x = buf[r, pl.ds(base + u * 16, 16)]
                        acc = acc + jnp.exp(x * _SCALE - _SCALE)
                    return acc

                o2_v[r] = lax.fori_loop(0, _PW // 64, _inner, acc)

            for r in range(8):
                y_r = y_all[g * 8 + r]
                relp = y_r - p * _PW
                start = pl.multiple_of(
                    jnp.clip((relp // 16) * 16, 0, _PW - 16), 16)
                v = buf[r, pl.ds(start, 16)]
                pick = jnp.where(lane == relp - start, v, 0.0)
                t2_v[r] = t2_v[r] + pick

        def _pair(q, carry):
            p0 = q * 2
            _copy(p0 + 1, x1_v, sem1).start()
            _copy(p0, x0_v, sem0).wait()
            _compute(x0_v, p0)

            @pl.when(p0 + 2 < _NP)
            def _():
                _copy(p0 + 2, x0_v, sem0).start()

            _copy(p0 + 1, x1_v, sem1).wait()
            _compute(x1_v, p0 + 1)
            return carry

        lax.fori_loop(0, _NP // 2, _pair, 0)
        _copy(_NP - 1, x0_v, sem0).wait()
        _compute(x0_v, _NP - 1)

        pltpu.sync_copy(o2_v, s_out.at[pl.ds(wid * 16 + g * 8, 8)])
        pltpu.sync_copy(t2_v, t_out.at[pl.ds(wid * 16 + g * 8, 8)])


def _sc_sumexp(y_hat, y, row0, rows_sc):
    b, n = y_hat.shape
    mesh = plsc.VectorSubcoreMesh(core_axis_name="c", subcore_axis_name="s")
    kfn = functools.partial(
        pl.kernel,
        mesh=mesh,
        out_type=(
            jax.ShapeDtypeStruct((rows_sc, 16), jnp.float32),
            jax.ShapeDtypeStruct((rows_sc, 16), jnp.float32),
        ),
        scratch_types=[
            pltpu.VMEM((8, _PW), jnp.float32),
            pltpu.VMEM((8, _PW), jnp.float32),
            pltpu.VMEM((16,), jnp.int32),
            pltpu.VMEM((8, 16), jnp.float32),
            pltpu.VMEM((8, 16), jnp.float32),
            pltpu.SemaphoreType.DMA,
            pltpu.SemaphoreType.DMA,
        ],
    )(functools.partial(_sc_sumexp_body, row0, n))
    return kfn(y_hat, y)


def _tc_body(y_ref, x_ref, s_out, t_out, acc_ref, t_ref, *, ncb, nclass, cb):
    j = pl.program_id(0)

    @pl.when(j == 0)
    def _init():
        acc_ref[...] = jnp.zeros_like(acc_ref)
        t_ref[...] = jnp.zeros_like(t_ref)

    x = x_ref[...]
    col = j * cb + lax.broadcasted_iota(jnp.int32, x.shape, 1)
    is_t = col == y_ref[...]
    t_ref[...] = t_ref[...] + jnp.sum(jnp.where(is_t, x, 0.0), axis=1,
                                      keepdims=True)
    z = jnp.where(col < nclass, x * _C1 - _C1, _NEG_HUGE)
    acc_ref[...] = acc_ref[...] + jnp.sum(jnp.exp2(z), axis=1, keepdims=True)

    @pl.when(j == ncb - 1)
    def _fin():
        s_out[...] = acc_ref[...]
        t_out[...] = t_ref[...]


def _tc_sumexp(y_hat, y, rows_tc, cb):
    b, n = y_hat.shape
    ncb = pl.cdiv(n, cb)
    return pl.pallas_call(
        functools.partial(_tc_body, ncb=ncb, nclass=n, cb=cb),
        grid=(ncb,),
        in_specs=[
            pl.BlockSpec((rows_tc, 1), lambda j: (0, 0)),
            pl.BlockSpec((rows_tc, cb), lambda j: (0, j)),
        ],
        out_specs=(
            pl.BlockSpec((rows_tc, 1), lambda j: (0, 0)),
            pl.BlockSpec((rows_tc, 1), lambda j: (0, 0)),
        ),
        out_shape=(
            jax.ShapeDtypeStruct((rows_tc, 1), jnp.float32),
            jax.ShapeDtypeStruct((rows_tc, 1), jnp.float32),
        ),
        scratch_shapes=[
            pltpu.VMEM((rows_tc, 1), jnp.float32),
            pltpu.VMEM((rows_tc, 1), jnp.float32),
        ],
    )(y[:rows_tc].reshape(rows_tc, 1), y_hat)


def _ep_body(stc_ref, ttc_ref, ssc_ref, tsc_ref, ysc_ref, tail_ref, out_ref,
             *, batch, ntail, ncov):
    tx = tail_ref[...]
    tcol = lax.broadcasted_iota(jnp.int32, tx.shape, 1)
    tz = jnp.where(tcol < ntail, tx * _C1 - _C1, _NEG_HUGE)
    tail = jnp.sum(jnp.exp2(tz), axis=1, keepdims=True)
    s_sc = jnp.sum(ssc_ref[...], axis=1, keepdims=True) + tail
    is_t = (ncov + tcol) == ysc_ref[...]
    t_tail = jnp.sum(jnp.where(is_t, tx, 0.0), axis=1, keepdims=True)
    t_sc = jnp.sum(tsc_ref[...], axis=1, keepdims=True) + t_tail
    s = jnp.concatenate([stc_ref[...], s_sc], axis=0)
    t = jnp.concatenate([ttc_ref[...], t_sc], axis=0)
    tm = _margined(t)
    e_t = jnp.exp2(t * _C1 - _C1)
    e_tm = jnp.exp2(tm * _C1 - _C1)
    s_mod = s - e_t + e_tm
    loss_rows = jnp.log(s_mod) + (_SCALE - _SCALE * tm)
    out_ref[...] = jnp.sum(loss_rows, axis=(0, 1), keepdims=True) / batch


def _epilogue(y_hat, y, s_tc, t_tc, s_sc_l, t_sc, b, n, rows_tc):
    rows_sc = b - rows_tc
    ntail = n - _NCOV
    out = pl.pallas_call(
        functools.partial(_ep_body, batch=b, ntail=ntail, ncov=_NCOV),
        grid=(1,),
        in_specs=[
            pl.BlockSpec((rows_tc, 1), lambda i: (0, 0)),
            pl.BlockSpec((rows_tc, 1), lambda i: (0, 0)),
            pl.BlockSpec((rows_sc, 16), lambda i: (0, 0)),
            pl.BlockSpec((rows_sc, 16), lambda i: (0, 0)),
            pl.BlockSpec((rows_sc, 1), lambda i: (1, 0)),
            pl.BlockSpec((rows_sc, 128), lambda i: (1, _NCOV // 128)),
        ],
        out_specs=pl.BlockSpec((1, 1), lambda i: (0, 0)),
        out_shape=jax.ShapeDtypeStruct((1, 1), jnp.float32),
    )(s_tc, t_tc, s_sc_l, t_sc, y.reshape(b, 1), y_hat)
    return out[0, 0]


@jax.jit
def kernel(y_hat, y):
    b, n = y_hat.shape
    rows_tc = b - _ROWS_SC
    s_sc_l, t_sc_l = _sc_sumexp(y_hat, y, rows_tc, _ROWS_SC)
    s_tc, t_tc = _tc_sumexp(y_hat, y, rows_tc, _CB)
    return _epilogue(y_hat, y, s_tc, t_tc, s_sc_l, t_sc_l, b, n, rows_tc)
